# baseline (speedup 1.0000x reference)
import functools
import math

import jax
import jax.numpy as jnp
from jax.experimental import pallas as pl
from jax.experimental.pallas import tpu as pltpu

E = 8
D = 768
H = 2 * D
T = 32768
TB = 2048
NB = T // TB

_bf16 = jnp.bfloat16
_f32 = jnp.float32


def _phase_a_kernel(x_ref, wr_ref, logits_ref, slots_ref, m_ref, l_ref, st_ref):
    i = pl.program_id(0)
    nb = pl.num_programs(0)
    x = x_ref[...]
    wr = wr_ref[...]
    logits = jax.lax.dot_general(
        x.astype(_bf16), wr.astype(_bf16),
        (((1,), (1,)), ((), ())), preferred_element_type=_f32)
    logits_ref[...] = logits

    @pl.when(i == 0)
    def _init():
        m_ref[...] = jnp.full_like(m_ref[...], -1e30)
        l_ref[...] = jnp.zeros_like(l_ref[...])
        st_ref[...] = jnp.zeros_like(st_ref[...])

    m_old = m_ref[...]
    bmax = jnp.max(logits, axis=0, keepdims=True)
    m_new = jnp.maximum(m_old, bmax)
    scale = jnp.exp(m_old - m_new)
    p = jnp.exp(logits - m_new)
    m_ref[...] = m_new
    l_ref[...] = l_ref[...] * scale + jnp.sum(p, axis=0, keepdims=True)
    st_ref[...] = st_ref[...] * scale + jax.lax.dot_general(
        x.astype(_bf16), p.astype(_bf16),
        (((0,), (0,)), ((), ())), preferred_element_type=_f32)

    @pl.when(i == nb - 1)
    def _fin():
        slots_ref[...] = (st_ref[...] / l_ref[...]).T


def _phase_b_kernel(slots_ref, w1_ref, b1_ref, g_ref, beta_ref, w2_ref, b2_ref,
                    out_ref):
    s = slots_ref[0]
    h = jax.lax.dot_general(
        s.astype(_bf16), w1_ref[0].astype(_bf16),
        (((1,), (1,)), ((), ())), preferred_element_type=_f32)
    h = h + b1_ref[0]
    mu = jnp.mean(h, axis=1, keepdims=True)
    var = jnp.mean((h - mu) ** 2, axis=1, keepdims=True)
    h = (h - mu) * jax.lax.rsqrt(var + 1e-5) * g_ref[0] + beta_ref[0]
    h = 0.5 * h * (1.0 + jax.lax.erf(h * (1.0 / math.sqrt(2.0))))
    out = jax.lax.dot_general(
        h.astype(_bf16), w2_ref[0].astype(_bf16),
        (((1,), (1,)), ((), ())), preferred_element_type=_f32)
    out_ref[0] = out + b2_ref[0]


def _phase_c_kernel(logits_ref, slots_ref, wr_ref, out_ref, loss_ref,
                    usage_ref):
    i = pl.program_id(0)
    nb = pl.num_programs(0)
    logits = logits_ref[...]
    mx = jnp.max(logits, axis=1, keepdims=True)
    p = jnp.exp(logits - mx)
    combine = p / jnp.sum(p, axis=1, keepdims=True)
    out_ref[...] = jax.lax.dot_general(
        combine.astype(_bf16), slots_ref[...].astype(_bf16),
        (((1,), (0,)), ((), ())), preferred_element_type=_f32)

    @pl.when(i == 0)
    def _init():
        usage_ref[...] = jnp.zeros_like(usage_ref[...])

    usage_ref[...] += jnp.sum(combine, axis=0, keepdims=True)

    @pl.when(i == nb - 1)
    def _fin():
        wr = wr_ref[...]
        rm = jnp.mean(wr, axis=1, keepdims=True)
        pe = jnp.exp(rm - jnp.max(rm, axis=0, keepdims=True))
        pe = pe / jnp.sum(pe, axis=0, keepdims=True)
        mean_usage = usage_ref[...] / float(T)
        loss = jax.lax.dot_general(
            mean_usage, pe, (((1,), (0,)), ((), ())),
            preferred_element_type=_f32)
        loss_ref[...] = float(E) * loss


@jax.jit
def kernel(x, Wr, W1, b1, g, beta, W2, b2):
    logits, slot_inputs = pl.pallas_call(
        _phase_a_kernel,
        grid=(NB,),
        in_specs=[
            pl.BlockSpec((TB, D), lambda i: (i, 0)),
            pl.BlockSpec((E, D), lambda i: (0, 0)),
        ],
        out_specs=[
            pl.BlockSpec((TB, E), lambda i: (i, 0)),
            pl.BlockSpec((E, D), lambda i: (0, 0)),
        ],
        out_shape=[
            jax.ShapeDtypeStruct((T, E), _f32),
            jax.ShapeDtypeStruct((E, D), _f32),
        ],
        scratch_shapes=[
            pltpu.VMEM((1, E), _f32),
            pltpu.VMEM((1, E), _f32),
            pltpu.VMEM((D, E), _f32),
        ],
    )(x, Wr)

    slot_outputs = pl.pallas_call(
        _phase_b_kernel,
        grid=(E,),
        in_specs=[
            pl.BlockSpec((1, 1, D), lambda e: (e, 0, 0)),
            pl.BlockSpec((1, H, D), lambda e: (e, 0, 0)),
            pl.BlockSpec((1, 1, H), lambda e: (e, 0, 0)),
            pl.BlockSpec((1, 1, H), lambda e: (e, 0, 0)),
            pl.BlockSpec((1, 1, H), lambda e: (e, 0, 0)),
            pl.BlockSpec((1, D, H), lambda e: (e, 0, 0)),
            pl.BlockSpec((1, 1, D), lambda e: (e, 0, 0)),
        ],
        out_specs=pl.BlockSpec((1, 1, D), lambda e: (e, 0, 0)),
        out_shape=jax.ShapeDtypeStruct((E, 1, D), _f32),
    )(slot_inputs.reshape(E, 1, D), W1, b1.reshape(E, 1, H),
      g.reshape(E, 1, H), beta.reshape(E, 1, H), W2, b2.reshape(E, 1, D))
    slot_outputs = slot_outputs.reshape(E, D)

    output, loss = pl.pallas_call(
        _phase_c_kernel,
        grid=(NB,),
        in_specs=[
            pl.BlockSpec((TB, E), lambda i: (i, 0)),
            pl.BlockSpec((E, D), lambda i: (0, 0)),
            pl.BlockSpec((E, D), lambda i: (0, 0)),
        ],
        out_specs=[
            pl.BlockSpec((TB, D), lambda i: (i, 0)),
            pl.BlockSpec((1, 1), lambda i: (0, 0)),
        ],
        out_shape=[
            jax.ShapeDtypeStruct((T, D), _f32),
            jax.ShapeDtypeStruct((1, 1), _f32),
        ],
        scratch_shapes=[
            pltpu.VMEM((1, E), _f32),
        ],
    )(logits, slot_outputs, Wr)

    return (output, loss.reshape(()))


# speedup vs baseline: 1.3735x; 1.3735x over previous
import math

import jax
import jax.numpy as jnp
from jax.experimental import pallas as pl
from jax.experimental.pallas import tpu as pltpu

E = 8
D = 768
H = 2 * D
T = 32768
TB = 2048
NB = T // TB

_bf16 = jnp.bfloat16
_f32 = jnp.float32


def _fused_kernel(x_ref, wr_ref, w1_ref, b1_ref, g_ref, beta_ref, w2_ref,
                  b2_ref, out_ref, loss_ref,
                  logits_s, m_s, l_s, s_s, so_s, usage_s):
    i = pl.program_id(0)

    @pl.when(i == 0)
    def _init():
        m_s[...] = jnp.full_like(m_s[...], -1e30)
        l_s[...] = jnp.zeros_like(l_s[...])
        s_s[...] = jnp.zeros_like(s_s[...])
        usage_s[...] = jnp.zeros_like(usage_s[...])

    @pl.when(i < NB)
    def _phase_a():
        x = x_ref[...]
        logits = jax.lax.dot_general(
            wr_ref[...].astype(_bf16), x.astype(_bf16),
            (((1,), (1,)), ((), ())), preferred_element_type=_f32)
        logits_s[i] = logits
        m_old = m_s[...]
        bmax = jnp.max(logits, axis=1, keepdims=True)
        m_new = jnp.maximum(m_old, bmax)
        scale = jnp.exp(m_old - m_new)
        p = jnp.exp(logits - m_new)
        m_s[...] = m_new
        l_s[...] = l_s[...] * scale + jnp.sum(p, axis=1, keepdims=True)
        s_s[...] = s_s[...] * scale + jax.lax.dot_general(
            p.astype(_bf16), x.astype(_bf16),
            (((1,), (0,)), ((), ())), preferred_element_type=_f32)

        @pl.when(i == NB - 1)
        def _finalize():
            s_s[...] = s_s[...] / l_s[...]

    @pl.when(jnp.logical_and(i >= NB, i < NB + E))
    def _phase_b():
        e = i - NB
        s = s_s[pl.ds(e, 1), :]
        h = jax.lax.dot_general(
            s.astype(_bf16), w1_ref[0].astype(_bf16),
            (((1,), (1,)), ((), ())), preferred_element_type=_f32)
        h = h + b1_ref[0]
        mu = jnp.mean(h, axis=1, keepdims=True)
        var = jnp.mean((h - mu) ** 2, axis=1, keepdims=True)
        h = (h - mu) * jax.lax.rsqrt(var + 1e-5) * g_ref[0] + beta_ref[0]
        h = 0.5 * h * (1.0 + jax.lax.erf(h * (1.0 / math.sqrt(2.0))))
        out = jax.lax.dot_general(
            h.astype(_bf16), w2_ref[0].astype(_bf16),
            (((1,), (1,)), ((), ())), preferred_element_type=_f32)
        so_s[pl.ds(e, 1), :] = out + b2_ref[0]

    @pl.when(i >= NB + E)
    def _phase_c():
        j = i - (NB + E)
        logits = logits_s[j]
        mx = jnp.max(logits, axis=0, keepdims=True)
        p = jnp.exp(logits - mx)
        combine = p / jnp.sum(p, axis=0, keepdims=True)
        out_ref[...] = jax.lax.dot_general(
            combine.astype(_bf16), so_s[...].astype(_bf16),
            (((0,), (0,)), ((), ())), preferred_element_type=_f32)
        usage_s[...] += jnp.sum(combine, axis=1, keepdims=True)

        @pl.when(i == NB + E + NB - 1)
        def _loss():
            rm = jnp.mean(wr_ref[...], axis=1, keepdims=True)
            pe = jnp.exp(rm - jnp.max(rm, axis=0, keepdims=True))
            pe = pe / jnp.sum(pe, axis=0, keepdims=True)
            mean_usage = usage_s[...] / float(T)
            loss_ref[...] = float(E) * jnp.sum(mean_usage * pe, keepdims=True)


@jax.jit
def kernel(x, Wr, W1, b1, g, beta, W2, b2):
    expert_idx = lambda i: (jnp.clip(i - NB, 0, E - 1), 0, 0)
    output, loss = pl.pallas_call(
        _fused_kernel,
        grid=(NB + E + NB,),
        in_specs=[
            pl.BlockSpec((TB, D), lambda i: (jnp.minimum(i, NB - 1), 0)),
            pl.BlockSpec((E, D), lambda i: (0, 0)),
            pl.BlockSpec((1, H, D), expert_idx),
            pl.BlockSpec((1, 1, H), expert_idx),
            pl.BlockSpec((1, 1, H), expert_idx),
            pl.BlockSpec((1, 1, H), expert_idx),
            pl.BlockSpec((1, D, H), expert_idx),
            pl.BlockSpec((1, 1, D), expert_idx),
        ],
        out_specs=[
            pl.BlockSpec((TB, D), lambda i: (jnp.maximum(i - (NB + E), 0), 0)),
            pl.BlockSpec((1, 1), lambda i: (0, 0)),
        ],
        out_shape=[
            jax.ShapeDtypeStruct((T, D), _f32),
            jax.ShapeDtypeStruct((1, 1), _f32),
        ],
        scratch_shapes=[
            pltpu.VMEM((NB, E, TB), _f32),
            pltpu.VMEM((E, 1), _f32),
            pltpu.VMEM((E, 1), _f32),
            pltpu.VMEM((E, D), _f32),
            pltpu.VMEM((E, D), _f32),
            pltpu.VMEM((E, 1), _f32),
        ],
    )(x, Wr, W1, b1.reshape(E, 1, H), g.reshape(E, 1, H),
      beta.reshape(E, 1, H), W2, b2.reshape(E, 1, D))

    return (output, loss.reshape(()))
